# baseline (speedup 1.0000x reference)
import functools

import jax
import jax.numpy as jnp
from jax import lax
from jax.experimental import pallas as pl
from jax.experimental.pallas import tpu as pltpu
from jax.experimental.pallas import tpu_sc as plsc

BATCH = 16384
FACTOR = 16
D_MLP = 64
NROWS = 1000000

_BW = 256
_NBLK = 3907
_MROWS = NROWS // 2
_GROWS = NROWS // 8

_info = plsc.get_sparse_core_info()
_NC = _info.num_cores
_NS = _info.num_subcores
_NW = _NC * _NS
_BPW = BATCH // _NW
_CH = 64
_NCH = _BPW // _CH


def _conv_mlp_body(xu_ref, xi_ref, e_ref, msk_ref, ou_ref, oi_ref):
    last = (pl.program_id(0) == _NBLK - 1).astype(jnp.float32)
    nt = (((1,), (1,)), ((), ()))
    e0 = e_ref[:_BW // 2, :]
    e1 = e_ref[_BW // 2:, :]
    for x_ref, o_ref in ((xu_ref, ou_ref), (xi_ref, oi_ref)):
        x = x_ref[...] * (1.0 - last * msk_ref[...])
        xb = x.astype(jnp.bfloat16)
        y0 = lax.dot_general(e0, xb, nt, preferred_element_type=jnp.float32)
        y1 = lax.dot_general(e1, xb, nt, preferred_element_type=jnp.float32)
        o_ref[...] = jnp.concatenate([y0, y1], axis=1)


def _conv_gmf_body(xu_ref, xi_ref, s_ref, msk_ref, ou_ref, oi_ref):
    last = (pl.program_id(0) == _NBLK - 1).astype(jnp.float32)
    nt = (((1,), (1,)), ((), ()))
    for x_ref, o_ref in ((xu_ref, ou_ref), (xi_ref, oi_ref)):
        x = x_ref[...] * (1.0 - last * msk_ref[...])
        xb = x.astype(jnp.bfloat16)
        ys = []
        for s in range(8):
            es = s_ref[s * (_BW // 8):(s + 1) * (_BW // 8), :]
            ys.append(lax.dot_general(es, xb, nt,
                                      preferred_element_type=jnp.float32))
        o_ref[...] = jnp.concatenate(ys, axis=1)


def _conv_mlp(xu, xi, e, msk):
    blk_in = pl.BlockSpec((D_MLP, _BW), lambda j: (0, j))
    rep = lambda j: (0, 0)
    blk_out = pl.BlockSpec((_BW // 2, 128), lambda j: (j, 0))
    return pl.pallas_call(
        _conv_mlp_body,
        grid=(_NBLK,),
        in_specs=[blk_in, blk_in,
                  pl.BlockSpec((_BW, _BW), rep),
                  pl.BlockSpec((D_MLP, _BW), rep)],
        out_specs=[blk_out, blk_out],
        out_shape=[jax.ShapeDtypeStruct((_MROWS, 128), jnp.float32)] * 2,
    )(xu, xi, e, msk)


def _conv_gmf(xu, xi, s, msk):
    blk_in = pl.BlockSpec((FACTOR, _BW), lambda j: (0, j))
    rep = lambda j: (0, 0)
    blk_out = pl.BlockSpec((_BW // 8, 128), lambda j: (j, 0))
    return pl.pallas_call(
        _conv_gmf_body,
        grid=(_NBLK,),
        in_specs=[blk_in, blk_in,
                  pl.BlockSpec((_BW, _BW), rep),
                  pl.BlockSpec((FACTOR, _BW), rep)],
        out_specs=[blk_out, blk_out],
        out_shape=[jax.ShapeDtypeStruct((_GROWS, 128), jnp.float32)] * 2,
    )(xu, xi, s, msk)


_sc_mesh = plsc.VectorSubcoreMesh(core_axis_name="c", subcore_axis_name="s")


@functools.partial(
    pl.kernel,
    mesh=_sc_mesh,
    compiler_params=pltpu.CompilerParams(use_tc_tiling_on_sc=True),
    out_type=[jax.ShapeDtypeStruct((BATCH, 128), jnp.float32)] * 4,
    scratch_types=[
        pltpu.VMEM((_BPW,), jnp.int32),
        pltpu.VMEM((_BPW,), jnp.int32),
        pltpu.VMEM((_BPW,), jnp.int32),
        pltpu.VMEM((_BPW,), jnp.int32),
        pltpu.VMEM((_CH, 128), jnp.float32),
        pltpu.VMEM((_CH, 128), jnp.float32),
        pltpu.VMEM((_CH, 128), jnp.float32),
        pltpu.VMEM((_CH, 128), jnp.float32),
        pltpu.SemaphoreType.DMA,
    ],
)
def _sc_gather(urow_m_hbm, irow_m_hbm, urow_g_hbm, irow_g_hbm,
               um_hbm, im_hbm, ug_hbm, ig_hbm,
               out_um, out_im, out_ug, out_ig,
               um_idx, im_idx, ug_idx, ig_idx,
               bum, bim, bug, big, sem):
    wid = lax.axis_index("s") * _NC + lax.axis_index("c")
    base = wid * _BPW
    sl = pl.ds(base, _BPW)
    pltpu.sync_copy(urow_m_hbm.at[sl], um_idx)
    pltpu.sync_copy(irow_m_hbm.at[sl], im_idx)
    pltpu.sync_copy(urow_g_hbm.at[sl], ug_idx)
    pltpu.sync_copy(irow_g_hbm.at[sl], ig_idx)

    def chunk(c, carry):
        off = c * _CH
        for g in range(_CH // 16):
            va = um_idx[pl.ds(off + g * 16, 16)]
            vb = im_idx[pl.ds(off + g * 16, 16)]
            vc = ug_idx[pl.ds(off + g * 16, 16)]
            vd = ig_idx[pl.ds(off + g * 16, 16)]
            for k in range(16):
                kk = g * 16 + k
                dst = pl.ds(kk, 1)
                pltpu.async_copy(um_hbm.at[pl.ds(va[k], 1)], bum.at[dst], sem)
                pltpu.async_copy(im_hbm.at[pl.ds(vb[k], 1)], bim.at[dst], sem)
                pltpu.async_copy(ug_hbm.at[pl.ds(vc[k], 1)], bug.at[dst], sem)
                pltpu.async_copy(ig_hbm.at[pl.ds(vd[k], 1)], big.at[dst], sem)
        pltpu.make_async_copy(um_hbm.at[pl.ds(0, _CH)], bum, sem).wait()
        pltpu.make_async_copy(im_hbm.at[pl.ds(0, _CH)], bim, sem).wait()
        pltpu.make_async_copy(ug_hbm.at[pl.ds(0, _CH)], bug, sem).wait()
        pltpu.make_async_copy(ig_hbm.at[pl.ds(0, _CH)], big, sem).wait()
        out_sl = pl.ds(base + off, _CH)
        pltpu.sync_copy(bum, out_um.at[out_sl])
        pltpu.sync_copy(bim, out_im.at[out_sl])
        pltpu.sync_copy(bug, out_ug.at[out_sl])
        pltpu.sync_copy(big, out_ig.at[out_sl])
        return carry

    lax.fori_loop(0, _NCH, chunk, 0)


_BB = 2048


def _tc_body(umw_ref, imw_ref, ugw_ref, igw_ref,
             mum_ref, mim_ref, mug_ref, mig_ref, rm_ref, rg_ref,
             w1u_ref, w1i_ref, b1_ref, w2_ref, b2_ref, w3_ref, b3_ref,
             wnm_ref, wng_ref, bn_ref, out_ref):
    um = jnp.dot(umw_ref[...] * mum_ref[...], rm_ref[...],
                 preferred_element_type=jnp.float32)
    im = jnp.dot(imw_ref[...] * mim_ref[...], rm_ref[...],
                 preferred_element_type=jnp.float32)
    ug = jnp.dot(ugw_ref[...] * mug_ref[...], rg_ref[...],
                 preferred_element_type=jnp.float32)
    ig = jnp.dot(igw_ref[...] * mig_ref[...], rg_ref[...],
                 preferred_element_type=jnp.float32)
    h = jnp.dot(um, w1u_ref[...], preferred_element_type=jnp.float32)
    h = h + jnp.dot(im, w1i_ref[...], preferred_element_type=jnp.float32)
    h = jnp.maximum(h + b1_ref[...], 0.0)
    h = jnp.maximum(
        jnp.dot(h, w2_ref[...], preferred_element_type=jnp.float32) + b2_ref[...], 0.0)
    m = jnp.maximum(
        jnp.dot(h, w3_ref[...], preferred_element_type=jnp.float32) + b3_ref[...], 0.0)
    g = ug * ig
    s_ = (jnp.dot(m, wnm_ref[...], preferred_element_type=jnp.float32)
          + jnp.dot(g, wng_ref[...], preferred_element_type=jnp.float32)
          + bn_ref[...])
    out_ref[...] = jax.nn.sigmoid(s_)


def _tc_mlp(umw, imw, ugw, igw, mum, mim, mug, mig, rm, rg,
            w1u, w1i, b1, w2, b2, w3, b3, wnm, wng, bn):
    grid = BATCH // _BB
    row = lambda i: (i, 0)
    rep = lambda i: (0, 0)
    wide = pl.BlockSpec((_BB, 128), row)
    return pl.pallas_call(
        _tc_body,
        grid=(grid,),
        in_specs=[
            wide, wide, wide, wide,
            wide, wide, wide, wide,
            pl.BlockSpec((128, D_MLP), rep),
            pl.BlockSpec((128, FACTOR), rep),
            pl.BlockSpec((D_MLP, D_MLP), rep),
            pl.BlockSpec((D_MLP, D_MLP), rep),
            pl.BlockSpec((1, D_MLP), rep),
            pl.BlockSpec((D_MLP, 32), rep),
            pl.BlockSpec((1, 32), rep),
            pl.BlockSpec((32, FACTOR), rep),
            pl.BlockSpec((1, FACTOR), rep),
            pl.BlockSpec((FACTOR, 1), rep),
            pl.BlockSpec((FACTOR, 1), rep),
            pl.BlockSpec((1, 1), rep),
        ],
        out_specs=pl.BlockSpec((_BB, 1), row),
        out_shape=jax.ShapeDtypeStruct((BATCH, 1), jnp.float32),
    )(umw, imw, ugw, igw, mum, mim, mug, mig, rm, rg,
      w1u, w1i, b1, w2, b2, w3, b3, wnm, wng, bn)


def kernel(user, item, user_embed_GMF, item_embed_GMF, user_embed_MLP,
           item_embed_MLP, W1, b1, W2, b2, W3, b3, Wn, bn):
    user = user.astype(jnp.int32)
    item = item.astype(jnp.int32)
    ar = jnp.arange(_BW, dtype=jnp.int32)
    e = (ar[None, :] == jnp.concatenate(
        [2 * jnp.arange(_BW // 2, dtype=jnp.int32),
         2 * jnp.arange(_BW // 2, dtype=jnp.int32) + 1])[:, None]
         ).astype(jnp.bfloat16)
    s8 = (ar[None, :] == jnp.concatenate(
        [8 * jnp.arange(_BW // 8, dtype=jnp.int32) + s for s in range(8)]
        )[:, None]).astype(jnp.bfloat16)
    oob = (ar >= 64).astype(jnp.float32)
    msk_m = jnp.broadcast_to(oob, (D_MLP, _BW))
    msk_g = jnp.broadcast_to(oob, (FACTOR, _BW))
    um128, im128 = _conv_mlp(user_embed_MLP.T, item_embed_MLP.T, e, msk_m)
    ug128, ig128 = _conv_gmf(user_embed_GMF.T, item_embed_GMF.T, s8, msk_g)
    umw, imw, ugw, igw = _sc_gather(
        user // 2, item // 2, user // 8, item // 8,
        um128, im128, ug128, ig128)
    l128 = jnp.arange(128, dtype=jnp.int32)
    mum = (l128[None, :] // 64 == (user % 2)[:, None]).astype(jnp.float32)
    mim = (l128[None, :] // 64 == (item % 2)[:, None]).astype(jnp.float32)
    mug = (l128[None, :] // 16 == (user % 8)[:, None]).astype(jnp.float32)
    mig = (l128[None, :] // 16 == (item % 8)[:, None]).astype(jnp.float32)
    rm = (l128[:, None] % 64 ==
          jnp.arange(D_MLP, dtype=jnp.int32)[None, :]).astype(jnp.float32)
    rg = (l128[:, None] % 16 ==
          jnp.arange(FACTOR, dtype=jnp.int32)[None, :]).astype(jnp.float32)
    w1u = W1[:, :D_MLP].T
    w1i = W1[:, D_MLP:].T
    wnm = Wn[:, :FACTOR].T
    wng = Wn[:, FACTOR:].T
    return _tc_mlp(umw, imw, ugw, igw, mum, mim, mug, mig, rm, rg,
                   w1u, w1i, b1.reshape(1, -1), W2.T, b2.reshape(1, -1),
                   W3.T, b3.reshape(1, -1), wnm, wng, bn.reshape(1, 1))


# speedup vs baseline: 2.7114x; 2.7114x over previous
import functools

import jax
import jax.numpy as jnp
from jax import lax
from jax.experimental import pallas as pl
from jax.experimental.pallas import tpu as pltpu
from jax.experimental.pallas import tpu_sc as plsc

BATCH = 16384
FACTOR = 16
D_MLP = 64
NROWS = 1000000

_BW = 1024
_NBLK = 977

_info = plsc.get_sparse_core_info()
_NC = _info.num_cores
_NS = _info.num_subcores
_NW = _NC * _NS
_BPW = BATCH // _NW
_CH = 64
_NCH = _BPW // _CH


def _tr_body(xu_ref, xi_ref, i_ref, ou_ref, oi_ref):
    tn = (((0,), (0,)), ((), ()))
    ident = i_ref[...]
    for x_ref, o_ref in ((xu_ref, ou_ref), (xi_ref, oi_ref)):
        xb = x_ref[...].astype(jnp.bfloat16)
        o_ref[...] = lax.dot_general(xb, ident, tn,
                                     preferred_element_type=jnp.float32)


def _transpose_pair(xu, xi, d):
    ident = jnp.eye(d, dtype=jnp.bfloat16)
    blk_in = pl.BlockSpec((d, _BW), lambda j: (0, j))
    blk_out = pl.BlockSpec((_BW, d), lambda j: (j, 0))
    return pl.pallas_call(
        _tr_body,
        grid=(_NBLK,),
        in_specs=[blk_in, blk_in, pl.BlockSpec((d, d), lambda j: (0, 0))],
        out_specs=[blk_out, blk_out],
        out_shape=[jax.ShapeDtypeStruct((NROWS, d), jnp.float32)] * 2,
    )(xu, xi, ident)


_sc_mesh = plsc.VectorSubcoreMesh(core_axis_name="c", subcore_axis_name="s")


@functools.partial(
    pl.kernel,
    mesh=_sc_mesh,
    compiler_params=pltpu.CompilerParams(use_tc_tiling_on_sc=True),
    out_type=[
        jax.ShapeDtypeStruct((BATCH, FACTOR), jnp.float32),
        jax.ShapeDtypeStruct((BATCH, FACTOR), jnp.float32),
        jax.ShapeDtypeStruct((BATCH, D_MLP), jnp.float32),
        jax.ShapeDtypeStruct((BATCH, D_MLP), jnp.float32),
    ],
    scratch_types=[
        pltpu.VMEM((_BPW,), jnp.int32),
        pltpu.VMEM((_BPW,), jnp.int32),
        pltpu.VMEM((_CH, FACTOR), jnp.float32),
        pltpu.VMEM((_CH, FACTOR), jnp.float32),
        pltpu.VMEM((_CH, D_MLP), jnp.float32),
        pltpu.VMEM((_CH, D_MLP), jnp.float32),
        pltpu.SemaphoreType.DMA,
    ],
)
def _sc_gather(user_hbm, item_hbm, ugmf_hbm, igmf_hbm, umlp_hbm, imlp_hbm,
               out_ug, out_ig, out_um, out_im,
               uidx, iidx, bug, big, bum, bim, sem):
    wid = lax.axis_index("s") * _NC + lax.axis_index("c")
    base = wid * _BPW
    pltpu.sync_copy(user_hbm.at[pl.ds(base, _BPW)], uidx)
    pltpu.sync_copy(item_hbm.at[pl.ds(base, _BPW)], iidx)

    def chunk(c, carry):
        off = c * _CH
        for g in range(_CH // 16):
            vu = uidx[pl.ds(off + g * 16, 16)]
            vi = iidx[pl.ds(off + g * 16, 16)]
            for k in range(16):
                iu = vu[k]
                ii = vi[k]
                kk = g * 16 + k
                pltpu.async_copy(ugmf_hbm.at[pl.ds(iu, 1)], bug.at[pl.ds(kk, 1)], sem)
                pltpu.async_copy(igmf_hbm.at[pl.ds(ii, 1)], big.at[pl.ds(kk, 1)], sem)
                pltpu.async_copy(umlp_hbm.at[pl.ds(iu, 1)], bum.at[pl.ds(kk, 1)], sem)
                pltpu.async_copy(imlp_hbm.at[pl.ds(ii, 1)], bim.at[pl.ds(kk, 1)], sem)
        pltpu.make_async_copy(ugmf_hbm.at[pl.ds(0, _CH)], bug, sem).wait()
        pltpu.make_async_copy(igmf_hbm.at[pl.ds(0, _CH)], big, sem).wait()
        pltpu.make_async_copy(umlp_hbm.at[pl.ds(0, _CH)], bum, sem).wait()
        pltpu.make_async_copy(imlp_hbm.at[pl.ds(0, _CH)], bim, sem).wait()
        out_sl = pl.ds(base + off, _CH)
        pltpu.sync_copy(bug, out_ug.at[out_sl])
        pltpu.sync_copy(big, out_ig.at[out_sl])
        pltpu.sync_copy(bum, out_um.at[out_sl])
        pltpu.sync_copy(bim, out_im.at[out_sl])
        return carry

    lax.fori_loop(0, _NCH, chunk, 0)


_BB = 2048


def _tc_body(ug_ref, ig_ref, um_ref, im_ref,
             w1u_ref, w1i_ref, b1_ref, w2_ref, b2_ref, w3_ref, b3_ref,
             wnm_ref, wng_ref, bn_ref, out_ref):
    h = jnp.dot(um_ref[...], w1u_ref[...], preferred_element_type=jnp.float32)
    h = h + jnp.dot(im_ref[...], w1i_ref[...], preferred_element_type=jnp.float32)
    h = jnp.maximum(h + b1_ref[...], 0.0)
    h = jnp.maximum(
        jnp.dot(h, w2_ref[...], preferred_element_type=jnp.float32) + b2_ref[...], 0.0)
    m = jnp.maximum(
        jnp.dot(h, w3_ref[...], preferred_element_type=jnp.float32) + b3_ref[...], 0.0)
    g = ug_ref[...] * ig_ref[...]
    s = (jnp.dot(m, wnm_ref[...], preferred_element_type=jnp.float32)
         + jnp.dot(g, wng_ref[...], preferred_element_type=jnp.float32)
         + bn_ref[...])
    out_ref[...] = jax.nn.sigmoid(s)


def _tc_mlp(ug, ig, um, im, w1u, w1i, b1, w2, b2, w3, b3, wnm, wng, bn):
    grid = BATCH // _BB
    row = lambda i: (i, 0)
    rep = lambda i: (0, 0)
    return pl.pallas_call(
        _tc_body,
        grid=(grid,),
        in_specs=[
            pl.BlockSpec((_BB, FACTOR), row),
            pl.BlockSpec((_BB, FACTOR), row),
            pl.BlockSpec((_BB, D_MLP), row),
            pl.BlockSpec((_BB, D_MLP), row),
            pl.BlockSpec((D_MLP, D_MLP), rep),
            pl.BlockSpec((D_MLP, D_MLP), rep),
            pl.BlockSpec((1, D_MLP), rep),
            pl.BlockSpec((D_MLP, 32), rep),
            pl.BlockSpec((1, 32), rep),
            pl.BlockSpec((32, FACTOR), rep),
            pl.BlockSpec((1, FACTOR), rep),
            pl.BlockSpec((FACTOR, 1), rep),
            pl.BlockSpec((FACTOR, 1), rep),
            pl.BlockSpec((1, 1), rep),
        ],
        out_specs=pl.BlockSpec((_BB, 1), row),
        out_shape=jax.ShapeDtypeStruct((BATCH, 1), jnp.float32),
    )(ug, ig, um, im, w1u, w1i, b1, w2, b2, w3, b3, wnm, wng, bn)


def kernel(user, item, user_embed_GMF, item_embed_GMF, user_embed_MLP,
           item_embed_MLP, W1, b1, W2, b2, W3, b3, Wn, bn):
    user = user.astype(jnp.int32)
    item = item.astype(jnp.int32)
    um_t, im_t = _transpose_pair(user_embed_MLP.T, item_embed_MLP.T, D_MLP)
    ug_t, ig_t = _transpose_pair(user_embed_GMF.T, item_embed_GMF.T, FACTOR)
    ug, ig, um, im = _sc_gather(user, item, ug_t, ig_t, um_t, im_t)
    w1u = W1[:, :D_MLP].T
    w1i = W1[:, D_MLP:].T
    wnm = Wn[:, :FACTOR].T
    wng = Wn[:, FACTOR:].T
    return _tc_mlp(ug, ig, um, im, w1u, w1i, b1.reshape(1, -1), W2.T,
                   b2.reshape(1, -1), W3.T, b3.reshape(1, -1), wnm, wng,
                   bn.reshape(1, 1))


# speedup vs baseline: 5.0026x; 1.8450x over previous
import functools

import jax
import jax.numpy as jnp
from jax import lax
from jax.experimental import pallas as pl
from jax.experimental.pallas import tpu as pltpu
from jax.experimental.pallas import tpu_sc as plsc

BATCH = 16384
FACTOR = 16
D_MLP = 64
NROWS = 1000000

_BW = 4096
_NBLK = 245

_info = plsc.get_sparse_core_info()
_NC = _info.num_cores
_NS = _info.num_subcores
_NW = _NC * _NS
_BPW = BATCH // _NW
_CH = 64
_NCH = _BPW // _CH


def _tr_body(xu_ref, xi_ref, i_ref, ou_ref, oi_ref):
    tn = (((0,), (0,)), ((), ()))
    ident = i_ref[...]
    for x_ref, o_ref in ((xu_ref, ou_ref), (xi_ref, oi_ref)):
        xb = x_ref[...].astype(jnp.bfloat16)
        o_ref[...] = lax.dot_general(xb, ident, tn,
                                     preferred_element_type=jnp.float32)


def _transpose_pair(xu, xi, d):
    ident = jnp.eye(d, dtype=jnp.bfloat16)
    blk_in = pl.BlockSpec((d, _BW), lambda j: (0, j))
    blk_out = pl.BlockSpec((_BW, d), lambda j: (j, 0))
    return pl.pallas_call(
        _tr_body,
        grid=(_NBLK,),
        in_specs=[blk_in, blk_in, pl.BlockSpec((d, d), lambda j: (0, 0))],
        out_specs=[blk_out, blk_out],
        out_shape=[jax.ShapeDtypeStruct((NROWS, d), jnp.float32)] * 2,
    )(xu, xi, ident)


_sc_mesh = plsc.VectorSubcoreMesh(core_axis_name="c", subcore_axis_name="s")


@functools.partial(
    pl.kernel,
    mesh=_sc_mesh,
    compiler_params=pltpu.CompilerParams(use_tc_tiling_on_sc=True),
    out_type=[
        jax.ShapeDtypeStruct((BATCH, FACTOR), jnp.float32),
        jax.ShapeDtypeStruct((BATCH, FACTOR), jnp.float32),
        jax.ShapeDtypeStruct((BATCH, D_MLP), jnp.float32),
        jax.ShapeDtypeStruct((BATCH, D_MLP), jnp.float32),
    ],
    scratch_types=[
        pltpu.VMEM((_BPW,), jnp.int32),
        pltpu.VMEM((_BPW,), jnp.int32),
        pltpu.VMEM((_CH, FACTOR), jnp.float32),
        pltpu.VMEM((_CH, FACTOR), jnp.float32),
        pltpu.VMEM((_CH, D_MLP), jnp.float32),
        pltpu.VMEM((_CH, D_MLP), jnp.float32),
        pltpu.SemaphoreType.DMA,
    ],
)
def _sc_gather(user_hbm, item_hbm, ugmf_hbm, igmf_hbm, umlp_hbm, imlp_hbm,
               out_ug, out_ig, out_um, out_im,
               uidx, iidx, bug, big, bum, bim, sem):
    wid = lax.axis_index("s") * _NC + lax.axis_index("c")
    base = wid * _BPW
    pltpu.sync_copy(user_hbm.at[pl.ds(base, _BPW)], uidx)
    pltpu.sync_copy(item_hbm.at[pl.ds(base, _BPW)], iidx)

    def chunk(c, carry):
        off = c * _CH
        for g in range(_CH // 16):
            vu = uidx[pl.ds(off + g * 16, 16)]
            vi = iidx[pl.ds(off + g * 16, 16)]
            for k in range(16):
                iu = vu[k]
                ii = vi[k]
                kk = g * 16 + k
                pltpu.async_copy(ugmf_hbm.at[pl.ds(iu, 1)], bug.at[pl.ds(kk, 1)], sem)
                pltpu.async_copy(igmf_hbm.at[pl.ds(ii, 1)], big.at[pl.ds(kk, 1)], sem)
                pltpu.async_copy(umlp_hbm.at[pl.ds(iu, 1)], bum.at[pl.ds(kk, 1)], sem)
                pltpu.async_copy(imlp_hbm.at[pl.ds(ii, 1)], bim.at[pl.ds(kk, 1)], sem)
        pltpu.make_async_copy(ugmf_hbm.at[pl.ds(0, _CH)], bug, sem).wait()
        pltpu.make_async_copy(igmf_hbm.at[pl.ds(0, _CH)], big, sem).wait()
        pltpu.make_async_copy(umlp_hbm.at[pl.ds(0, _CH)], bum, sem).wait()
        pltpu.make_async_copy(imlp_hbm.at[pl.ds(0, _CH)], bim, sem).wait()
        out_sl = pl.ds(base + off, _CH)
        pltpu.sync_copy(bug, out_ug.at[out_sl])
        pltpu.sync_copy(big, out_ig.at[out_sl])
        pltpu.sync_copy(bum, out_um.at[out_sl])
        pltpu.sync_copy(bim, out_im.at[out_sl])
        return carry

    lax.fori_loop(0, _NCH, chunk, 0)


_BB = 2048


def _tc_body(ug_ref, ig_ref, um_ref, im_ref,
             w1u_ref, w1i_ref, b1_ref, w2_ref, b2_ref, w3_ref, b3_ref,
             wnm_ref, wng_ref, bn_ref, out_ref):
    h = jnp.dot(um_ref[...], w1u_ref[...], preferred_element_type=jnp.float32)
    h = h + jnp.dot(im_ref[...], w1i_ref[...], preferred_element_type=jnp.float32)
    h = jnp.maximum(h + b1_ref[...], 0.0)
    h = jnp.maximum(
        jnp.dot(h, w2_ref[...], preferred_element_type=jnp.float32) + b2_ref[...], 0.0)
    m = jnp.maximum(
        jnp.dot(h, w3_ref[...], preferred_element_type=jnp.float32) + b3_ref[...], 0.0)
    g = ug_ref[...] * ig_ref[...]
    s = (jnp.dot(m, wnm_ref[...], preferred_element_type=jnp.float32)
         + jnp.dot(g, wng_ref[...], preferred_element_type=jnp.float32)
         + bn_ref[...])
    out_ref[...] = jax.nn.sigmoid(s)


def _tc_mlp(ug, ig, um, im, w1u, w1i, b1, w2, b2, w3, b3, wnm, wng, bn):
    grid = BATCH // _BB
    row = lambda i: (i, 0)
    rep = lambda i: (0, 0)
    return pl.pallas_call(
        _tc_body,
        grid=(grid,),
        in_specs=[
            pl.BlockSpec((_BB, FACTOR), row),
            pl.BlockSpec((_BB, FACTOR), row),
            pl.BlockSpec((_BB, D_MLP), row),
            pl.BlockSpec((_BB, D_MLP), row),
            pl.BlockSpec((D_MLP, D_MLP), rep),
            pl.BlockSpec((D_MLP, D_MLP), rep),
            pl.BlockSpec((1, D_MLP), rep),
            pl.BlockSpec((D_MLP, 32), rep),
            pl.BlockSpec((1, 32), rep),
            pl.BlockSpec((32, FACTOR), rep),
            pl.BlockSpec((1, FACTOR), rep),
            pl.BlockSpec((FACTOR, 1), rep),
            pl.BlockSpec((FACTOR, 1), rep),
            pl.BlockSpec((1, 1), rep),
        ],
        out_specs=pl.BlockSpec((_BB, 1), row),
        out_shape=jax.ShapeDtypeStruct((BATCH, 1), jnp.float32),
    )(ug, ig, um, im, w1u, w1i, b1, w2, b2, w3, b3, wnm, wng, bn)


def kernel(user, item, user_embed_GMF, item_embed_GMF, user_embed_MLP,
           item_embed_MLP, W1, b1, W2, b2, W3, b3, Wn, bn):
    user = user.astype(jnp.int32)
    item = item.astype(jnp.int32)
    um_t, im_t = _transpose_pair(user_embed_MLP.T, item_embed_MLP.T, D_MLP)
    ug_t, ig_t = _transpose_pair(user_embed_GMF.T, item_embed_GMF.T, FACTOR)
    ug, ig, um, im = _sc_gather(user, item, ug_t, ig_t, um_t, im_t)
    w1u = W1[:, :D_MLP].T
    w1i = W1[:, D_MLP:].T
    wnm = Wn[:, :FACTOR].T
    wng = Wn[:, FACTOR:].T
    return _tc_mlp(ug, ig, um, im, w1u, w1i, b1.reshape(1, -1), W2.T,
                   b2.reshape(1, -1), W3.T, b3.reshape(1, -1), wnm, wng,
                   bn.reshape(1, 1))


# speedup vs baseline: 5.4500x; 1.0894x over previous
import functools

import jax
import jax.numpy as jnp
from jax import lax
from jax.experimental import pallas as pl
from jax.experimental.pallas import tpu as pltpu
from jax.experimental.pallas import tpu_sc as plsc

BATCH = 16384
FACTOR = 16
D_MLP = 64
NROWS = 1000000

_BW = 8192
_NBLK = 123

_info = plsc.get_sparse_core_info()
_NC = _info.num_cores
_NS = _info.num_subcores
_NW = _NC * _NS
_BPW = BATCH // _NW
_CH = 64
_NCH = _BPW // _CH


def _tr_body(xu_ref, xi_ref, i_ref, ou_ref, oi_ref):
    tn = (((0,), (0,)), ((), ()))
    ident = i_ref[...]
    for x_ref, o_ref in ((xu_ref, ou_ref), (xi_ref, oi_ref)):
        xb = x_ref[...].astype(jnp.bfloat16)
        o_ref[...] = lax.dot_general(xb, ident, tn,
                                     preferred_element_type=jnp.float32)


def _transpose_pair(xu, xi, d):
    ident = jnp.eye(d, dtype=jnp.bfloat16)
    blk_in = pl.BlockSpec((d, _BW), lambda j: (0, j))
    blk_out = pl.BlockSpec((_BW, d), lambda j: (j, 0))
    return pl.pallas_call(
        _tr_body,
        grid=(_NBLK,),
        in_specs=[blk_in, blk_in, pl.BlockSpec((d, d), lambda j: (0, 0))],
        out_specs=[blk_out, blk_out],
        out_shape=[jax.ShapeDtypeStruct((NROWS, d), jnp.float32)] * 2,
    )(xu, xi, ident)


_sc_mesh = plsc.VectorSubcoreMesh(core_axis_name="c", subcore_axis_name="s")


@functools.partial(
    pl.kernel,
    mesh=_sc_mesh,
    compiler_params=pltpu.CompilerParams(use_tc_tiling_on_sc=True),
    out_type=[
        jax.ShapeDtypeStruct((BATCH, FACTOR), jnp.float32),
        jax.ShapeDtypeStruct((BATCH, FACTOR), jnp.float32),
        jax.ShapeDtypeStruct((BATCH, D_MLP), jnp.float32),
        jax.ShapeDtypeStruct((BATCH, D_MLP), jnp.float32),
    ],
    scratch_types=[
        pltpu.VMEM((_BPW,), jnp.int32),
        pltpu.VMEM((_BPW,), jnp.int32),
        pltpu.VMEM((_CH, FACTOR), jnp.float32),
        pltpu.VMEM((_CH, FACTOR), jnp.float32),
        pltpu.VMEM((_CH, D_MLP), jnp.float32),
        pltpu.VMEM((_CH, D_MLP), jnp.float32),
        pltpu.SemaphoreType.DMA,
    ],
)
def _sc_gather(user_hbm, item_hbm, ugmf_hbm, igmf_hbm, umlp_hbm, imlp_hbm,
               out_ug, out_ig, out_um, out_im,
               uidx, iidx, bug, big, bum, bim, sem):
    wid = lax.axis_index("s") * _NC + lax.axis_index("c")
    base = wid * _BPW
    pltpu.sync_copy(user_hbm.at[pl.ds(base, _BPW)], uidx)
    pltpu.sync_copy(item_hbm.at[pl.ds(base, _BPW)], iidx)

    def chunk(c, carry):
        off = c * _CH
        for g in range(_CH // 16):
            vu = uidx[pl.ds(off + g * 16, 16)]
            vi = iidx[pl.ds(off + g * 16, 16)]
            for k in range(16):
                iu = vu[k]
                ii = vi[k]
                kk = g * 16 + k
                pltpu.async_copy(ugmf_hbm.at[pl.ds(iu, 1)], bug.at[pl.ds(kk, 1)], sem)
                pltpu.async_copy(igmf_hbm.at[pl.ds(ii, 1)], big.at[pl.ds(kk, 1)], sem)
                pltpu.async_copy(umlp_hbm.at[pl.ds(iu, 1)], bum.at[pl.ds(kk, 1)], sem)
                pltpu.async_copy(imlp_hbm.at[pl.ds(ii, 1)], bim.at[pl.ds(kk, 1)], sem)
        pltpu.make_async_copy(ugmf_hbm.at[pl.ds(0, _CH)], bug, sem).wait()
        pltpu.make_async_copy(igmf_hbm.at[pl.ds(0, _CH)], big, sem).wait()
        pltpu.make_async_copy(umlp_hbm.at[pl.ds(0, _CH)], bum, sem).wait()
        pltpu.make_async_copy(imlp_hbm.at[pl.ds(0, _CH)], bim, sem).wait()
        out_sl = pl.ds(base + off, _CH)
        pltpu.sync_copy(bug, out_ug.at[out_sl])
        pltpu.sync_copy(big, out_ig.at[out_sl])
        pltpu.sync_copy(bum, out_um.at[out_sl])
        pltpu.sync_copy(bim, out_im.at[out_sl])
        return carry

    lax.fori_loop(0, _NCH, chunk, 0)


_BB = 2048


def _tc_body(ug_ref, ig_ref, um_ref, im_ref,
             w1u_ref, w1i_ref, b1_ref, w2_ref, b2_ref, w3_ref, b3_ref,
             wnm_ref, wng_ref, bn_ref, out_ref):
    h = jnp.dot(um_ref[...], w1u_ref[...], preferred_element_type=jnp.float32)
    h = h + jnp.dot(im_ref[...], w1i_ref[...], preferred_element_type=jnp.float32)
    h = jnp.maximum(h + b1_ref[...], 0.0)
    h = jnp.maximum(
        jnp.dot(h, w2_ref[...], preferred_element_type=jnp.float32) + b2_ref[...], 0.0)
    m = jnp.maximum(
        jnp.dot(h, w3_ref[...], preferred_element_type=jnp.float32) + b3_ref[...], 0.0)
    g = ug_ref[...] * ig_ref[...]
    s = (jnp.dot(m, wnm_ref[...], preferred_element_type=jnp.float32)
         + jnp.dot(g, wng_ref[...], preferred_element_type=jnp.float32)
         + bn_ref[...])
    out_ref[...] = jax.nn.sigmoid(s)


def _tc_mlp(ug, ig, um, im, w1u, w1i, b1, w2, b2, w3, b3, wnm, wng, bn):
    grid = BATCH // _BB
    row = lambda i: (i, 0)
    rep = lambda i: (0, 0)
    return pl.pallas_call(
        _tc_body,
        grid=(grid,),
        in_specs=[
            pl.BlockSpec((_BB, FACTOR), row),
            pl.BlockSpec((_BB, FACTOR), row),
            pl.BlockSpec((_BB, D_MLP), row),
            pl.BlockSpec((_BB, D_MLP), row),
            pl.BlockSpec((D_MLP, D_MLP), rep),
            pl.BlockSpec((D_MLP, D_MLP), rep),
            pl.BlockSpec((1, D_MLP), rep),
            pl.BlockSpec((D_MLP, 32), rep),
            pl.BlockSpec((1, 32), rep),
            pl.BlockSpec((32, FACTOR), rep),
            pl.BlockSpec((1, FACTOR), rep),
            pl.BlockSpec((FACTOR, 1), rep),
            pl.BlockSpec((FACTOR, 1), rep),
            pl.BlockSpec((1, 1), rep),
        ],
        out_specs=pl.BlockSpec((_BB, 1), row),
        out_shape=jax.ShapeDtypeStruct((BATCH, 1), jnp.float32),
    )(ug, ig, um, im, w1u, w1i, b1, w2, b2, w3, b3, wnm, wng, bn)


def kernel(user, item, user_embed_GMF, item_embed_GMF, user_embed_MLP,
           item_embed_MLP, W1, b1, W2, b2, W3, b3, Wn, bn):
    user = user.astype(jnp.int32)
    item = item.astype(jnp.int32)
    um_t, im_t = _transpose_pair(user_embed_MLP.T, item_embed_MLP.T, D_MLP)
    ug_t, ig_t = _transpose_pair(user_embed_GMF.T, item_embed_GMF.T, FACTOR)
    ug, ig, um, im = _sc_gather(user, item, ug_t, ig_t, um_t, im_t)
    w1u = W1[:, :D_MLP].T
    w1i = W1[:, D_MLP:].T
    wnm = Wn[:, :FACTOR].T
    wng = Wn[:, FACTOR:].T
    return _tc_mlp(ug, ig, um, im, w1u, w1i, b1.reshape(1, -1), W2.T,
                   b2.reshape(1, -1), W3.T, b3.reshape(1, -1), wnm, wng,
                   bn.reshape(1, 1))


# speedup vs baseline: 5.5805x; 1.0239x over previous
import functools

import jax
import jax.numpy as jnp
from jax import lax
from jax.experimental import pallas as pl
from jax.experimental.pallas import tpu as pltpu
from jax.experimental.pallas import tpu_sc as plsc

BATCH = 16384
FACTOR = 16
D_MLP = 64
NROWS = 1000000

_BW = 16384
_NBLK = 62

_info = plsc.get_sparse_core_info()
_NC = _info.num_cores
_NS = _info.num_subcores
_NW = _NC * _NS
_BPW = BATCH // _NW
_CH = 64
_NCH = _BPW // _CH


def _tr_body(xu_ref, xi_ref, i_ref, ou_ref, oi_ref):
    tn = (((0,), (0,)), ((), ()))
    ident = i_ref[...]
    for x_ref, o_ref in ((xu_ref, ou_ref), (xi_ref, oi_ref)):
        xb = x_ref[...].astype(jnp.bfloat16)
        o_ref[...] = lax.dot_general(xb, ident, tn,
                                     preferred_element_type=jnp.float32)


def _transpose_pair(xu, xi, d):
    ident = jnp.eye(d, dtype=jnp.bfloat16)
    blk_in = pl.BlockSpec((d, _BW), lambda j: (0, j))
    blk_out = pl.BlockSpec((_BW, d), lambda j: (j, 0))
    return pl.pallas_call(
        _tr_body,
        grid=(_NBLK,),
        in_specs=[blk_in, blk_in, pl.BlockSpec((d, d), lambda j: (0, 0))],
        out_specs=[blk_out, blk_out],
        out_shape=[jax.ShapeDtypeStruct((NROWS, d), jnp.float32)] * 2,
    )(xu, xi, ident)


_sc_mesh = plsc.VectorSubcoreMesh(core_axis_name="c", subcore_axis_name="s")


@functools.partial(
    pl.kernel,
    mesh=_sc_mesh,
    compiler_params=pltpu.CompilerParams(use_tc_tiling_on_sc=True),
    out_type=[
        jax.ShapeDtypeStruct((BATCH, FACTOR), jnp.float32),
        jax.ShapeDtypeStruct((BATCH, FACTOR), jnp.float32),
        jax.ShapeDtypeStruct((BATCH, D_MLP), jnp.float32),
        jax.ShapeDtypeStruct((BATCH, D_MLP), jnp.float32),
    ],
    scratch_types=[
        pltpu.VMEM((_BPW,), jnp.int32),
        pltpu.VMEM((_BPW,), jnp.int32),
        pltpu.VMEM((_CH, FACTOR), jnp.float32),
        pltpu.VMEM((_CH, FACTOR), jnp.float32),
        pltpu.VMEM((_CH, D_MLP), jnp.float32),
        pltpu.VMEM((_CH, D_MLP), jnp.float32),
        pltpu.SemaphoreType.DMA,
    ],
)
def _sc_gather(user_hbm, item_hbm, ugmf_hbm, igmf_hbm, umlp_hbm, imlp_hbm,
               out_ug, out_ig, out_um, out_im,
               uidx, iidx, bug, big, bum, bim, sem):
    wid = lax.axis_index("s") * _NC + lax.axis_index("c")
    base = wid * _BPW
    pltpu.sync_copy(user_hbm.at[pl.ds(base, _BPW)], uidx)
    pltpu.sync_copy(item_hbm.at[pl.ds(base, _BPW)], iidx)

    def chunk(c, carry):
        off = c * _CH
        for g in range(_CH // 16):
            vu = uidx[pl.ds(off + g * 16, 16)]
            vi = iidx[pl.ds(off + g * 16, 16)]
            for k in range(16):
                iu = vu[k]
                ii = vi[k]
                kk = g * 16 + k
                pltpu.async_copy(ugmf_hbm.at[pl.ds(iu, 1)], bug.at[pl.ds(kk, 1)], sem)
                pltpu.async_copy(igmf_hbm.at[pl.ds(ii, 1)], big.at[pl.ds(kk, 1)], sem)
                pltpu.async_copy(umlp_hbm.at[pl.ds(iu, 1)], bum.at[pl.ds(kk, 1)], sem)
                pltpu.async_copy(imlp_hbm.at[pl.ds(ii, 1)], bim.at[pl.ds(kk, 1)], sem)
        pltpu.make_async_copy(ugmf_hbm.at[pl.ds(0, _CH)], bug, sem).wait()
        pltpu.make_async_copy(igmf_hbm.at[pl.ds(0, _CH)], big, sem).wait()
        pltpu.make_async_copy(umlp_hbm.at[pl.ds(0, _CH)], bum, sem).wait()
        pltpu.make_async_copy(imlp_hbm.at[pl.ds(0, _CH)], bim, sem).wait()
        out_sl = pl.ds(base + off, _CH)
        pltpu.sync_copy(bug, out_ug.at[out_sl])
        pltpu.sync_copy(big, out_ig.at[out_sl])
        pltpu.sync_copy(bum, out_um.at[out_sl])
        pltpu.sync_copy(bim, out_im.at[out_sl])
        return carry

    lax.fori_loop(0, _NCH, chunk, 0)


_BB = 2048


def _tc_body(ug_ref, ig_ref, um_ref, im_ref,
             w1u_ref, w1i_ref, b1_ref, w2_ref, b2_ref, w3_ref, b3_ref,
             wnm_ref, wng_ref, bn_ref, out_ref):
    h = jnp.dot(um_ref[...], w1u_ref[...], preferred_element_type=jnp.float32)
    h = h + jnp.dot(im_ref[...], w1i_ref[...], preferred_element_type=jnp.float32)
    h = jnp.maximum(h + b1_ref[...], 0.0)
    h = jnp.maximum(
        jnp.dot(h, w2_ref[...], preferred_element_type=jnp.float32) + b2_ref[...], 0.0)
    m = jnp.maximum(
        jnp.dot(h, w3_ref[...], preferred_element_type=jnp.float32) + b3_ref[...], 0.0)
    g = ug_ref[...] * ig_ref[...]
    s = (jnp.dot(m, wnm_ref[...], preferred_element_type=jnp.float32)
         + jnp.dot(g, wng_ref[...], preferred_element_type=jnp.float32)
         + bn_ref[...])
    out_ref[...] = jax.nn.sigmoid(s)


def _tc_mlp(ug, ig, um, im, w1u, w1i, b1, w2, b2, w3, b3, wnm, wng, bn):
    grid = BATCH // _BB
    row = lambda i: (i, 0)
    rep = lambda i: (0, 0)
    return pl.pallas_call(
        _tc_body,
        grid=(grid,),
        in_specs=[
            pl.BlockSpec((_BB, FACTOR), row),
            pl.BlockSpec((_BB, FACTOR), row),
            pl.BlockSpec((_BB, D_MLP), row),
            pl.BlockSpec((_BB, D_MLP), row),
            pl.BlockSpec((D_MLP, D_MLP), rep),
            pl.BlockSpec((D_MLP, D_MLP), rep),
            pl.BlockSpec((1, D_MLP), rep),
            pl.BlockSpec((D_MLP, 32), rep),
            pl.BlockSpec((1, 32), rep),
            pl.BlockSpec((32, FACTOR), rep),
            pl.BlockSpec((1, FACTOR), rep),
            pl.BlockSpec((FACTOR, 1), rep),
            pl.BlockSpec((FACTOR, 1), rep),
            pl.BlockSpec((1, 1), rep),
        ],
        out_specs=pl.BlockSpec((_BB, 1), row),
        out_shape=jax.ShapeDtypeStruct((BATCH, 1), jnp.float32),
    )(ug, ig, um, im, w1u, w1i, b1, w2, b2, w3, b3, wnm, wng, bn)


def kernel(user, item, user_embed_GMF, item_embed_GMF, user_embed_MLP,
           item_embed_MLP, W1, b1, W2, b2, W3, b3, Wn, bn):
    user = user.astype(jnp.int32)
    item = item.astype(jnp.int32)
    um_t, im_t = _transpose_pair(user_embed_MLP.T, item_embed_MLP.T, D_MLP)
    ug_t, ig_t = _transpose_pair(user_embed_GMF.T, item_embed_GMF.T, FACTOR)
    ug, ig, um, im = _sc_gather(user, item, ug_t, ig_t, um_t, im_t)
    w1u = W1[:, :D_MLP].T
    w1i = W1[:, D_MLP:].T
    wnm = Wn[:, :FACTOR].T
    wng = Wn[:, FACTOR:].T
    return _tc_mlp(ug, ig, um, im, w1u, w1i, b1.reshape(1, -1), W2.T,
                   b2.reshape(1, -1), W3.T, b3.reshape(1, -1), wnm, wng,
                   bn.reshape(1, 1))
